# baseline (speedup 1.0000x reference)
import functools

import jax
import jax.numpy as jnp
from jax import lax
from jax.experimental import pallas as pl
from jax.experimental.pallas import tpu as pltpu
from jax.experimental.pallas import tpu_sc as plsc

N_INPUT = 1000
N_HIDDEN = 64
N_LATENT = 32
B = 1024

NC = 2
NS = 16
NW = NC * NS
EPW = B // NW
NCHUNK = N_HIDDEN // 16
NLANE = 16


def _mask_kernel(m_ref, x_ref, xl_ref):
    m = m_ref[...]
    cols = jax.lax.broadcasted_iota(jnp.int32, (B, N_INPUT), 1)
    hit = jnp.any(m == cols, axis=0, keepdims=True)
    keep = jnp.where(hit, 0.0, 1.0).astype(jnp.float32)
    xl_ref[...] = jnp.log1p(x_ref[...] * keep)


def _masked_log1p(x, mi):
    return pl.pallas_call(
        _mask_kernel,
        out_shape=jax.ShapeDtypeStruct((B, N_INPUT), jnp.float32),
    )(mi.reshape(B, 1), x)


def _sc_gmv_body(mi_hbm, table_hbm, xl_hbm, h16_hbm,
                 m_v, xl_v, abuf, hb2):
    wid = lax.axis_index("s") * NC + lax.axis_index("c")
    base = wid * EPW
    pltpu.sync_copy(mi_hbm.at[pl.ds(base, EPW)], m_v.at[pl.ds(0, EPW)])
    pltpu.sync_copy(xl_hbm.at[pl.ds(base, EPW)], xl_v)

    lane = lax.iota(jnp.int32, 16)

    def example_body(e, carry):
        g = m_v[pl.ds(e, 16)][0]
        for jc in range(NCHUNK):
            pltpu.sync_copy(table_hbm.at[g, jc], abuf.at[jc])

            def kc_body(kc, accs):
                xlc = xl_v[e, pl.ds(kc * 16, 16)]
                return tuple(
                    accs[j] + abuf[jc, j, pl.ds(kc * 16, 16)] * xlc
                    for j in range(16))

            accs = lax.fori_loop(
                0, 62, kc_body,
                tuple(jnp.zeros((16,), jnp.float32) for _ in range(16)),
                unroll=2)
            xlt = jnp.where(lane >= 8, xl_v[e, pl.ds(984, 16)], 0.0)
            accs = tuple(accs[j] + abuf[jc, j, pl.ds(984, 16)] * xlt
                         for j in range(16))

            for j in range(16):
                hb2[jc * 16 + j, :] = accs[j]

        pltpu.sync_copy(hb2, h16_hbm.at[base + e])
        return carry

    lax.fori_loop(0, EPW, example_body, 0, unroll=False)


def _gather_matvec(mi, table4, xl):
    mesh = plsc.VectorSubcoreMesh(core_axis_name="c", subcore_axis_name="s",
                                  num_cores=NC, num_subcores=NS)
    run = functools.partial(
        pl.kernel,
        out_type=jax.ShapeDtypeStruct((B, N_HIDDEN, NLANE), jnp.float32),
        mesh=mesh,
        compiler_params=pltpu.CompilerParams(use_tc_tiling_on_sc=False),
        scratch_types=[
            pltpu.VMEM((EPW + 16, ), jnp.int32),
            pltpu.VMEM((EPW, N_INPUT), jnp.float32),
            pltpu.VMEM((NCHUNK, 16, N_INPUT), jnp.float32),
            pltpu.VMEM((N_HIDDEN, NLANE), jnp.float32),
        ],
    )(_sc_gmv_body)
    return run(mi, table4, xl)


def _tail_kernel(m_ref, h16_ref, bt_ref, W1_ref, b1_ref, ls_ref, lb_ref,
                 W3_ref, b3_ref, W4_ref, b4_ref, mean_ref, scale_ref):
    rrows = jax.lax.broadcasted_iota(jnp.int32, (N_HIDDEN * NLANE, N_HIDDEN), 0)
    rcols = jax.lax.broadcasted_iota(jnp.int32, (N_HIDDEN * NLANE, N_HIDDEN), 1)
    R = (rrows // NLANE == rcols).astype(jnp.float32)
    h = jnp.dot(h16_ref[...], R, preferred_element_type=jnp.float32)
    cols = jax.lax.broadcasted_iota(jnp.int32, (B, N_INPUT), 1)
    oh = (m_ref[...] == cols).astype(jnp.float32)
    bv = jnp.dot(oh, bt_ref[...], preferred_element_type=jnp.float32)
    h = h + bv
    z = jnp.dot(h, W1_ref[...], preferred_element_type=jnp.float32) + b1_ref[...]
    mu = jnp.mean(z, axis=1, keepdims=True)
    var = jnp.mean((z - mu) ** 2, axis=1, keepdims=True)
    z = (z - mu) * jax.lax.rsqrt(var + 1e-6) * ls_ref[...] + lb_ref[...]
    z = jnp.maximum(z, 0.0)
    mean_ref[...] = jnp.dot(z, W3_ref[...], preferred_element_type=jnp.float32) + b3_ref[...]
    lv = jnp.dot(z, W4_ref[...], preferred_element_type=jnp.float32) + b4_ref[...]
    scale_ref[...] = jnp.exp(lv)


def _tail(mi, h16, bvecs_table, W1, b1, ln_scale, ln_bias, W3, b3, W4, b4):
    return pl.pallas_call(
        _tail_kernel,
        out_shape=(jax.ShapeDtypeStruct((B, N_LATENT), jnp.float32),
                   jax.ShapeDtypeStruct((B, N_LATENT), jnp.float32)),
    )(mi.reshape(B, 1), h16, bvecs_table, W1, b1.reshape(1, N_HIDDEN),
      ln_scale.reshape(1, N_HIDDEN), ln_bias.reshape(1, N_HIDDEN),
      W3, b3.reshape(1, N_LATENT), W4, b4.reshape(1, N_LATENT))


def kernel(x, masked_genes, amats_table, bvecs_table, W1, b1, ln_scale,
           ln_bias, W3, b3, W4, b4):
    mi = masked_genes.astype(jnp.int32)
    xl = _masked_log1p(x, mi)
    table4 = amats_table.reshape(N_INPUT, NCHUNK, 16, N_INPUT)
    h16 = _gather_matvec(mi, table4, xl)
    return _tail(mi, h16.reshape(B, N_HIDDEN * NLANE), bvecs_table, W1, b1,
                 ln_scale, ln_bias, W3, b3, W4, b4)


# speedup vs baseline: 1.1288x; 1.1288x over previous
import functools

import jax
import jax.numpy as jnp
from jax import lax
from jax.experimental import pallas as pl
from jax.experimental.pallas import tpu as pltpu
from jax.experimental.pallas import tpu_sc as plsc

N_INPUT = 1000
N_HIDDEN = 64
N_LATENT = 32
B = 1024

NC = 2
NS = 16
NW = NC * NS
EPW = B // NW
NCHUNK = N_HIDDEN // 16
NLANE = 16


def _mask_kernel(m_ref, x_ref, xl_ref):
    m = m_ref[...]
    cols = jax.lax.broadcasted_iota(jnp.int32, (B, N_INPUT), 1)
    hit = jnp.any(m == cols, axis=0, keepdims=True)
    keep = jnp.where(hit, 0.0, 1.0).astype(jnp.float32)
    xl_ref[...] = jnp.log1p(x_ref[...] * keep)


def _masked_log1p(x, mi):
    return pl.pallas_call(
        _mask_kernel,
        out_shape=jax.ShapeDtypeStruct((B, N_INPUT), jnp.float32),
    )(mi.reshape(B, 1), x)


def _sc_gmv_body(mi_hbm, table_hbm, xl_hbm, h16_hbm,
                 m_v, xl_v, abuf, hb2):
    wid = lax.axis_index("s") * NC + lax.axis_index("c")
    base = wid * EPW
    pltpu.sync_copy(mi_hbm.at[pl.ds(base, EPW)], m_v.at[pl.ds(0, EPW)])
    pltpu.sync_copy(xl_hbm.at[pl.ds(base, EPW)], xl_v)

    lane = lax.iota(jnp.int32, 16)

    def example_body(e, carry):
        g = m_v[pl.ds(e, 16)][0]
        pltpu.sync_copy(table_hbm.at[g], abuf)
        for jc in range(NCHUNK):

            def kc_body(kc, accs):
                xlc = xl_v[e, pl.ds(kc * 16, 16)]
                return tuple(
                    accs[j] + abuf[jc * 16 + j, pl.ds(kc * 16, 16)] * xlc
                    for j in range(16))

            accs = lax.fori_loop(
                0, 62, kc_body,
                tuple(jnp.zeros((16,), jnp.float32) for _ in range(16)),
                unroll=2)
            xlt = jnp.where(lane >= 8, xl_v[e, pl.ds(984, 16)], 0.0)
            accs = tuple(accs[j] + abuf[jc * 16 + j, pl.ds(984, 16)] * xlt
                         for j in range(16))

            for j in range(16):
                hb2[jc * 16 + j, :] = accs[j]

        pltpu.sync_copy(hb2, h16_hbm.at[base + e])
        return carry

    lax.fori_loop(0, EPW, example_body, 0, unroll=False)


def _gather_matvec(mi, table4, xl):
    mesh = plsc.VectorSubcoreMesh(core_axis_name="c", subcore_axis_name="s",
                                  num_cores=NC, num_subcores=NS)
    run = functools.partial(
        pl.kernel,
        out_type=jax.ShapeDtypeStruct((B, N_HIDDEN, NLANE), jnp.float32),
        mesh=mesh,
        compiler_params=pltpu.CompilerParams(use_tc_tiling_on_sc=False),
        scratch_types=[
            pltpu.VMEM((EPW + 16, ), jnp.int32),
            pltpu.VMEM((EPW, N_INPUT), jnp.float32),
            pltpu.VMEM((N_HIDDEN, N_INPUT), jnp.float32),
            pltpu.VMEM((N_HIDDEN, NLANE), jnp.float32),
        ],
    )(_sc_gmv_body)
    return run(mi, table4, xl)


def _tail_kernel(m_ref, h16_ref, bt_ref, W1_ref, b1_ref, ls_ref, lb_ref,
                 W3_ref, b3_ref, W4_ref, b4_ref, mean_ref, scale_ref):
    rrows = jax.lax.broadcasted_iota(jnp.int32, (N_HIDDEN * NLANE, N_HIDDEN), 0)
    rcols = jax.lax.broadcasted_iota(jnp.int32, (N_HIDDEN * NLANE, N_HIDDEN), 1)
    R = (rrows // NLANE == rcols).astype(jnp.float32)
    h = jnp.dot(h16_ref[...], R, preferred_element_type=jnp.float32)
    cols = jax.lax.broadcasted_iota(jnp.int32, (B, N_INPUT), 1)
    oh = (m_ref[...] == cols).astype(jnp.float32)
    bv = jnp.dot(oh, bt_ref[...], preferred_element_type=jnp.float32)
    h = h + bv
    z = jnp.dot(h, W1_ref[...], preferred_element_type=jnp.float32) + b1_ref[...]
    mu = jnp.mean(z, axis=1, keepdims=True)
    var = jnp.mean((z - mu) ** 2, axis=1, keepdims=True)
    z = (z - mu) * jax.lax.rsqrt(var + 1e-6) * ls_ref[...] + lb_ref[...]
    z = jnp.maximum(z, 0.0)
    mean_ref[...] = jnp.dot(z, W3_ref[...], preferred_element_type=jnp.float32) + b3_ref[...]
    lv = jnp.dot(z, W4_ref[...], preferred_element_type=jnp.float32) + b4_ref[...]
    scale_ref[...] = jnp.exp(lv)


def _tail(mi, h16, bvecs_table, W1, b1, ln_scale, ln_bias, W3, b3, W4, b4):
    return pl.pallas_call(
        _tail_kernel,
        out_shape=(jax.ShapeDtypeStruct((B, N_LATENT), jnp.float32),
                   jax.ShapeDtypeStruct((B, N_LATENT), jnp.float32)),
    )(mi.reshape(B, 1), h16, bvecs_table, W1, b1.reshape(1, N_HIDDEN),
      ln_scale.reshape(1, N_HIDDEN), ln_bias.reshape(1, N_HIDDEN),
      W3, b3.reshape(1, N_LATENT), W4, b4.reshape(1, N_LATENT))


def kernel(x, masked_genes, amats_table, bvecs_table, W1, b1, ln_scale,
           ln_bias, W3, b3, W4, b4):
    mi = masked_genes.astype(jnp.int32)
    xl = _masked_log1p(x, mi)
    table3 = amats_table.reshape(N_INPUT, N_HIDDEN, N_INPUT)
    h16 = _gather_matvec(mi, table3, xl)
    return _tail(mi, h16.reshape(B, N_HIDDEN * NLANE), bvecs_table, W1, b1,
                 ln_scale, ln_bias, W3, b3, W4, b4)
